# Initial kernel scaffold; baseline (speedup 1.0000x reference)
import jax
import jax.numpy as jnp
from jax.experimental import pallas as pl


def kernel(x, segment_mask, token_table, pos_table, seg_table):
    raise NotImplementedError("write your pallas kernel here")


# speedup vs baseline: 3.6294x; 3.6294x over previous
import functools

import jax
import jax.numpy as jnp
from jax import lax
from jax.experimental import pallas as pl
from jax.experimental.pallas import tpu as pltpu
from jax.experimental.pallas import tpu_sc as plsc

DIM = 64
SEQ = 200
BATCH = 1024
N = BATCH * SEQ
LANES = 16

NC, NS = 2, 16
NW = NC * NS
RPW = N // NW
CHUNK = 128
NCH = RPW // CHUNK


def _prep_body(segmask_ref, pos_ref, seg_ref, cidx_ref, comb_ref):
    s = segmask_ref[...]
    p = lax.broadcasted_iota(jnp.int32, (BATCH, SEQ), 1)
    cidx_ref[...] = s * SEQ + p
    pos = pos_ref[0:SEQ, :]
    comb_ref[0:SEQ, :] = pos + seg_ref[0:1, :]
    comb_ref[SEQ:2 * SEQ, :] = pos + seg_ref[1:2, :]


def _prep(segment_mask, pos_table, seg_table):
    return pl.pallas_call(
        _prep_body,
        out_shape=(
            jax.ShapeDtypeStruct((BATCH, SEQ), jnp.int32),
            jax.ShapeDtypeStruct((2 * SEQ, DIM), jnp.float32),
        ),
    )(segment_mask, pos_table, seg_table)


def _sc_body(x_hbm, cidx_hbm, tok_hbm, comb_hbm, out_hbm, xv, cv, tv, mv, gsem):
    cid = lax.axis_index("c")
    sid = lax.axis_index("s")
    wid = sid * NC + cid
    base_blk = wid * NCH

    def chunk(g, carry):
        blk = base_blk + g
        pltpu.sync_copy(x_hbm.at[blk], xv)
        pltpu.sync_copy(cidx_hbm.at[blk], cv)
        pltpu.async_copy(tok_hbm.at[xv], tv, gsem).wait()
        pltpu.async_copy(comb_hbm.at[cv], mv, gsem).wait()

        def row(r, c2):
            for j in range(DIM // LANES):
                sl = pl.ds(j * LANES, LANES)
                tv[r, sl] = tv[r, sl] + mv[r, sl]
            return c2

        lax.fori_loop(0, CHUNK, row, 0)
        pltpu.sync_copy(tv, out_hbm.at[pl.ds(blk * CHUNK, CHUNK)])
        return carry

    lax.fori_loop(0, NCH, chunk, 0)


@functools.partial(jax.jit, static_argnames=())
def _sc_lookup(x2, cidx2, token_table, comb):
    mesh = plsc.VectorSubcoreMesh(
        core_axis_name="c", subcore_axis_name="s", num_cores=NC, num_subcores=NS
    )
    f = pl.kernel(
        _sc_body,
        out_type=jax.ShapeDtypeStruct((N, DIM), jnp.float32),
        mesh=mesh,
        scratch_types=[
            pltpu.VMEM((CHUNK,), jnp.int32),
            pltpu.VMEM((CHUNK,), jnp.int32),
            pltpu.VMEM((CHUNK, DIM), jnp.float32),
            pltpu.VMEM((CHUNK, DIM), jnp.float32),
            pltpu.SemaphoreType.DMA,
        ],
        compiler_params=pltpu.CompilerParams(use_tc_tiling_on_sc=False),
    )
    return f(x2, cidx2, token_table, comb)


def kernel(x, segment_mask, token_table, pos_table, seg_table):
    x = x.astype(jnp.int32)
    segment_mask = segment_mask.astype(jnp.int32)
    cidx, comb = _prep(segment_mask, pos_table, seg_table)
    x2 = x.reshape(N // CHUNK, CHUNK)
    cidx2 = cidx.reshape(N // CHUNK, CHUNK)
    out = _sc_lookup(x2, cidx2, token_table, comb)
    return out.reshape(BATCH, SEQ, DIM)


# speedup vs baseline: 4.4097x; 1.2150x over previous
import functools

import jax
import jax.numpy as jnp
from jax import lax
from jax.experimental import pallas as pl
from jax.experimental.pallas import tpu as pltpu
from jax.experimental.pallas import tpu_sc as plsc

DIM = 64
SEQ = 200
BATCH = 1024
N = BATCH * SEQ
LANES = 16

NC, NS = 2, 16
NW = NC * NS
RPW = N // NW
CHUNK = 128
NCH = RPW // CHUNK


def _prep_body(segmask_ref, pos_ref, seg_ref, cidx_ref, comb_ref):
    s = segmask_ref[...]
    p = lax.broadcasted_iota(jnp.int32, (BATCH, SEQ), 1)
    cidx_ref[...] = s * SEQ + p
    pos = pos_ref[0:SEQ, :]
    comb_ref[0:SEQ, :] = pos + seg_ref[0:1, :]
    comb_ref[SEQ:2 * SEQ, :] = pos + seg_ref[1:2, :]


def _prep(segment_mask, pos_table, seg_table):
    return pl.pallas_call(
        _prep_body,
        out_shape=(
            jax.ShapeDtypeStruct((BATCH, SEQ), jnp.int32),
            jax.ShapeDtypeStruct((2 * SEQ, DIM), jnp.float32),
        ),
    )(segment_mask, pos_table, seg_table)


NB = 4


def _sc_body(x_hbm, cidx_hbm, tok_hbm, comb_hbm, out_hbm, xv, cv, tv, mv,
             isem, gsem, osem):
    cid = lax.axis_index("c")
    sid = lax.axis_index("s")
    wid = sid * NC + cid
    base_blk = wid * NCH


    def stage_a(g, slot, may_drain):
        blk = base_blk + g
        if may_drain:
            def _drain():
                pltpu.make_async_copy(
                    tv.at[slot], out_hbm.at[pl.ds(blk * CHUNK, CHUNK)],
                    osem.at[slot]).wait()
            pl.when(g >= NB)(_drain)
        pltpu.async_copy(x_hbm.at[blk], xv.at[slot], isem.at[slot])
        pltpu.async_copy(cidx_hbm.at[blk], cv.at[slot], isem.at[slot])

    def stage_b(g, slot):
        blk = base_blk + g
        pltpu.make_async_copy(x_hbm.at[blk], xv.at[slot], isem.at[slot]).wait()
        pltpu.make_async_copy(cidx_hbm.at[blk], cv.at[slot], isem.at[slot]).wait()
        pltpu.async_copy(tok_hbm.at[xv.at[slot]], tv.at[slot], gsem.at[slot])
        pltpu.async_copy(comb_hbm.at[cv.at[slot]], mv.at[slot], gsem.at[slot])

    def stage_c(g, slot):
        blk = base_blk + g
        pltpu.make_async_copy(
            tok_hbm.at[pl.ds(0, CHUNK)], tv.at[slot], gsem.at[slot]).wait()
        pltpu.make_async_copy(
            comb_hbm.at[pl.ds(0, CHUNK)], mv.at[slot], gsem.at[slot]).wait()

        def row(r, c2):
            for j in range(DIM // LANES):
                sl = pl.ds(j * LANES, LANES)
                tv[slot, r, sl] = tv[slot, r, sl] + mv[slot, r, sl]
            return c2

        lax.fori_loop(0, CHUNK, row, 0)
        pltpu.async_copy(tv.at[slot], out_hbm.at[pl.ds(blk * CHUNK, CHUNK)],
                         osem.at[slot])

    stage_a(0, 0, False)
    stage_a(1, 1, False)
    stage_b(0, 0)

    def outer(g0, carry):
        for b in range(NB):
            g = g0 + b
            stage_c(g, b)
            stage_b(g + 1, (b + 1) % NB)
            stage_a(g + 2, (b + 2) % NB, True)
        return carry

    lax.fori_loop(0, (NCH - 2) // NB, lambda i, c: outer(i * NB, c), 0,
                  unroll=False)

    stage_b(NCH - 1, (NCH - 1) % NB)
    stage_c(NCH - 2, (NCH - 2) % NB)
    stage_c(NCH - 1, (NCH - 1) % NB)
    for g in range(NCH - NB, NCH):
        blk = base_blk + g
        pltpu.make_async_copy(
            tv.at[g % NB], out_hbm.at[pl.ds(blk * CHUNK, CHUNK)],
            osem.at[g % NB]).wait()


@functools.partial(jax.jit, static_argnames=())
def _sc_lookup(x2, cidx2, token_table, comb):
    mesh = plsc.VectorSubcoreMesh(
        core_axis_name="c", subcore_axis_name="s", num_cores=NC, num_subcores=NS
    )
    f = pl.kernel(
        _sc_body,
        out_type=jax.ShapeDtypeStruct((N, DIM), jnp.float32),
        mesh=mesh,
        scratch_types=[
            pltpu.VMEM((NB, CHUNK), jnp.int32),
            pltpu.VMEM((NB, CHUNK), jnp.int32),
            pltpu.VMEM((NB, CHUNK, DIM), jnp.float32),
            pltpu.VMEM((NB, CHUNK, DIM), jnp.float32),
            pltpu.SemaphoreType.DMA((NB,)),
            pltpu.SemaphoreType.DMA((NB,)),
            pltpu.SemaphoreType.DMA((NB,)),
        ],
        compiler_params=pltpu.CompilerParams(use_tc_tiling_on_sc=False),
    )
    return f(x2, cidx2, token_table, comb)


def kernel(x, segment_mask, token_table, pos_table, seg_table):
    x = x.astype(jnp.int32)
    segment_mask = segment_mask.astype(jnp.int32)
    cidx, comb = _prep(segment_mask, pos_table, seg_table)
    x2 = x.reshape(N // CHUNK, CHUNK)
    cidx2 = cidx.reshape(N // CHUNK, CHUNK)
    out = _sc_lookup(x2, cidx2, token_table, comb)
    return out.reshape(BATCH, SEQ, DIM)
